# baseline (speedup 1.0000x reference)
import functools

import jax
import jax.numpy as jnp
from jax import lax
from jax.experimental import pallas as pl
from jax.experimental.pallas import tpu as pltpu
from jax.experimental.pallas import tpu_sc as plsc

BM = 256
BK = 2048
BIG = 2 ** 30


def _argmin_body(x_ref, w_ref, ones_ref, idx_ref, dev_ref, wn_ref):
    i = pl.program_id(0)
    k_total = w_ref.shape[0]

    @pl.when(i == 0)
    def _():
        w = w_ref[...]
        wn_ref[...] = jnp.sum(w ** 2, axis=-1).reshape(1, k_total)

    x = x_ref[...]
    xn = jnp.sum(x ** 2, axis=-1, keepdims=True)
    x2 = x + x
    best = jnp.full((BM, 1), jnp.float32(jnp.inf))
    bidx = jnp.full((BM, 1), BIG, dtype=jnp.int32)
    for t in range(k_total // BK):
        wt = w_ref[t * BK:(t + 1) * BK, :]
        mm2 = lax.dot_general(x2, wt, (((1,), (1,)), ((), ())),
                              preferred_element_type=jnp.float32)
        wn = wn_ref[:, t * BK:(t + 1) * BK]
        s = (xn + wn) - mm2
        mn = jnp.min(s, axis=1, keepdims=True)
        io = lax.broadcasted_iota(jnp.int32, (BM, BK), 1) + t * BK
        ci = jnp.min(jnp.where(s == mn, io, jnp.int32(BIG)), axis=1,
                     keepdims=True)
        upd = mn < best
        best = jnp.where(upd, mn, best)
        bidx = jnp.where(upd, ci, bidx)
    idx_ref[0, 0, :] = bidx[:, 0]
    dev_ref[0, 0, :] = best[:, 0]


def _argmin_call(x2, w):
    m, d = x2.shape
    k = w.shape[0]
    nblk = m // BM
    ones = jnp.ones((1, d), jnp.float32)
    return pl.pallas_call(
        _argmin_body,
        grid=(nblk,),
        in_specs=[
            pl.BlockSpec((BM, d), lambda i: (i, 0)),
            pl.BlockSpec((k, d), lambda i: (0, 0)),
            pl.BlockSpec((1, d), lambda i: (0, 0)),
        ],
        out_specs=[
            pl.BlockSpec((1, 1, BM), lambda i: (i, 0, 0)),
            pl.BlockSpec((1, 1, BM), lambda i: (i, 0, 0)),
        ],
        out_shape=[
            jax.ShapeDtypeStruct((nblk, 1, BM), jnp.int32),
            jax.ShapeDtypeStruct((nblk, 1, BM), jnp.float32),
        ],
        scratch_shapes=[
            pltpu.VMEM((1, k), jnp.float32),
        ],
    )(x2, w, ones)


def _make_gather(v, d, b):
    info = plsc.get_sparse_core_info()
    nw = info.num_cores * info.num_subcores
    b_per_w = b // nw
    mesh = plsc.VectorSubcoreMesh(core_axis_name="c", subcore_axis_name="s")

    @functools.partial(
        pl.kernel, mesh=mesh,
        out_type=jax.ShapeDtypeStruct((b, d), jnp.float32),
        scratch_types=[
            pltpu.VMEM((b_per_w,), jnp.int32),
            pltpu.VMEM((b_per_w, d), jnp.float32),
            pltpu.SemaphoreType.DMA,
        ],
    )
    def gather(table_hbm, idx_hbm, out_hbm, idx_v, rows_v, sem):
        wid = lax.axis_index("s") * info.num_cores + lax.axis_index("c")
        base = wid * b_per_w
        pltpu.sync_copy(idx_hbm.at[pl.ds(base, b_per_w)], idx_v)
        pltpu.async_copy(table_hbm.at[idx_v], rows_v, sem).wait()
        pltpu.sync_copy(rows_v, out_hbm.at[pl.ds(base, b_per_w)])

    return gather


def kernel(x, W):
    b, hw, d = x.shape
    kk = W.shape[0]
    m = b * hw
    x2 = x.reshape(m, d)
    idx_blk, dev_blk = _argmin_call(x2, W)
    idx = idx_blk.reshape(m)
    symbols = _make_gather(kk, d, m)(W, idx)
    out = symbols.reshape(b, hw, d)
    deviation = (jnp.sum(dev_blk) / jnp.float32(m * d)) * jnp.float32(1.25)
    return (out, deviation)


# speedup vs baseline: 1.1070x; 1.1070x over previous
import functools

import jax
import jax.numpy as jnp
from jax import lax
from jax.experimental import pallas as pl
from jax.experimental.pallas import tpu as pltpu
from jax.experimental.pallas import tpu_sc as plsc

BM = 1024
BK = 2048
BIG = 2 ** 30


def _argmin_body(x_ref, w_ref, ones_ref, idx_ref, dev_ref, wn_ref):
    i = pl.program_id(0)
    k_total = w_ref.shape[0]

    @pl.when(i == 0)
    def _():
        w = w_ref[...]
        wn_ref[...] = jnp.sum(w ** 2, axis=-1).reshape(1, k_total)

    x = x_ref[...]
    xn = jnp.sum(x ** 2, axis=-1, keepdims=True)
    x2 = x + x
    best = jnp.full((BM, 1), jnp.float32(jnp.inf))
    bidx = jnp.full((BM, 1), BIG, dtype=jnp.int32)
    for t in range(k_total // BK):
        wt = w_ref[t * BK:(t + 1) * BK, :]
        mm2 = lax.dot_general(x2, wt, (((1,), (1,)), ((), ())),
                              preferred_element_type=jnp.float32)
        wn = wn_ref[:, t * BK:(t + 1) * BK]
        s = (xn + wn) - mm2
        mn = jnp.min(s, axis=1, keepdims=True)
        io = lax.broadcasted_iota(jnp.int32, (BM, BK), 1) + t * BK
        ci = jnp.min(jnp.where(s == mn, io, jnp.int32(BIG)), axis=1,
                     keepdims=True)
        upd = mn < best
        best = jnp.where(upd, mn, best)
        bidx = jnp.where(upd, ci, bidx)
    idx_ref[0, 0, :] = bidx[:, 0]
    dev_ref[0, 0, :] = best[:, 0]


def _argmin_call(x2, w):
    m, d = x2.shape
    k = w.shape[0]
    nblk = m // BM
    ones = jnp.ones((1, d), jnp.float32)
    return pl.pallas_call(
        _argmin_body,
        grid=(nblk,),
        in_specs=[
            pl.BlockSpec((BM, d), lambda i: (i, 0)),
            pl.BlockSpec((k, d), lambda i: (0, 0)),
            pl.BlockSpec((1, d), lambda i: (0, 0)),
        ],
        out_specs=[
            pl.BlockSpec((1, 1, BM), lambda i: (i, 0, 0)),
            pl.BlockSpec((1, 1, BM), lambda i: (i, 0, 0)),
        ],
        out_shape=[
            jax.ShapeDtypeStruct((nblk, 1, BM), jnp.int32),
            jax.ShapeDtypeStruct((nblk, 1, BM), jnp.float32),
        ],
        scratch_shapes=[
            pltpu.VMEM((1, k), jnp.float32),
        ],
    )(x2, w, ones)


def _make_gather(v, d, b):
    info = plsc.get_sparse_core_info()
    nw = info.num_cores * info.num_subcores
    b_per_w = b // nw
    mesh = plsc.VectorSubcoreMesh(core_axis_name="c", subcore_axis_name="s")

    @functools.partial(
        pl.kernel, mesh=mesh,
        out_type=jax.ShapeDtypeStruct((b, d), jnp.float32),
        scratch_types=[
            pltpu.VMEM((b_per_w,), jnp.int32),
            pltpu.VMEM((b_per_w, d), jnp.float32),
            pltpu.SemaphoreType.DMA,
        ],
    )
    def gather(table_hbm, idx_hbm, out_hbm, idx_v, rows_v, sem):
        wid = lax.axis_index("s") * info.num_cores + lax.axis_index("c")
        base = wid * b_per_w
        pltpu.sync_copy(idx_hbm.at[pl.ds(base, b_per_w)], idx_v)
        pltpu.async_copy(table_hbm.at[idx_v], rows_v, sem).wait()
        pltpu.sync_copy(rows_v, out_hbm.at[pl.ds(base, b_per_w)])

    return gather


def kernel(x, W):
    b, hw, d = x.shape
    kk = W.shape[0]
    m = b * hw
    x2 = x.reshape(m, d)
    idx_blk, dev_blk = _argmin_call(x2, W)
    idx = idx_blk.reshape(m)
    symbols = _make_gather(kk, d, m)(W, idx)
    out = symbols.reshape(b, hw, d)
    deviation = (jnp.sum(dev_blk) / jnp.float32(m * d)) * jnp.float32(1.25)
    return (out, deviation)


# speedup vs baseline: 1.1075x; 1.0005x over previous
import functools

import jax
import jax.numpy as jnp
from jax import lax
from jax.experimental import pallas as pl
from jax.experimental.pallas import tpu as pltpu
from jax.experimental.pallas import tpu_sc as plsc

BM = 1024
BK = 2048
BIG = 2 ** 30


def _argmin_body(x_ref, w_ref, ones_ref, idx_ref, dev_ref, wn_ref):
    i = pl.program_id(0)
    k_total = w_ref.shape[0]

    @pl.when(i == 0)
    def _():
        w = w_ref[...]
        wn_ref[...] = jnp.sum(w ** 2, axis=-1).reshape(1, k_total)

    x = x_ref[...]
    xn = jnp.sum(x ** 2, axis=-1, keepdims=True)
    x2 = x + x
    best = jnp.full((BM, 1), jnp.float32(jnp.inf))
    bidx = jnp.full((BM, 1), BIG, dtype=jnp.int32)
    for t in range(k_total // BK):
        wt = w_ref[t * BK:(t + 1) * BK, :]
        mm2 = lax.dot_general(x2, wt, (((1,), (1,)), ((), ())),
                              preferred_element_type=jnp.float32)
        wn = wn_ref[:, t * BK:(t + 1) * BK]
        s = (xn + wn) - mm2
        mn = jnp.min(s, axis=1, keepdims=True)
        io = lax.broadcasted_iota(jnp.int32, (BM, BK), 1) + t * BK
        ci = jnp.min(jnp.where(s == mn, io, jnp.int32(BIG)), axis=1,
                     keepdims=True)
        upd = mn < best
        best = jnp.where(upd, mn, best)
        bidx = jnp.where(upd, ci, bidx)
    idx_ref[0, :] = bidx[:, 0]
    dev_ref[0, :] = best[:, 0]


def _argmin_call(x2, w):
    m, d = x2.shape
    k = w.shape[0]
    nblk = m // BM
    ones = jnp.ones((1, d), jnp.float32)
    return pl.pallas_call(
        _argmin_body,
        grid=(nblk,),
        in_specs=[
            pl.BlockSpec((BM, d), lambda i: (i, 0)),
            pl.BlockSpec((k, d), lambda i: (0, 0)),
            pl.BlockSpec((1, d), lambda i: (0, 0)),
        ],
        out_specs=[
            pl.BlockSpec((1, BM), lambda i: (0, i)),
            pl.BlockSpec((1, BM), lambda i: (0, i)),
        ],
        out_shape=[
            jax.ShapeDtypeStruct((1, m), jnp.int32),
            jax.ShapeDtypeStruct((1, m), jnp.float32),
        ],
        scratch_shapes=[
            pltpu.VMEM((1, k), jnp.float32),
        ],
    )(x2, w, ones)


def _make_gather(v, d, b):
    info = plsc.get_sparse_core_info()
    nw = info.num_cores * info.num_subcores
    b_per_w = b // nw
    mesh = plsc.VectorSubcoreMesh(core_axis_name="c", subcore_axis_name="s")

    @functools.partial(
        pl.kernel, mesh=mesh,
        out_type=jax.ShapeDtypeStruct((b, d), jnp.float32),
        scratch_types=[
            pltpu.VMEM((b_per_w,), jnp.int32),
            pltpu.VMEM((b_per_w, d), jnp.float32),
            pltpu.SemaphoreType.DMA,
        ],
    )
    def gather(table_hbm, idx_hbm, out_hbm, idx_v, rows_v, sem):
        wid = lax.axis_index("s") * info.num_cores + lax.axis_index("c")
        base = wid * b_per_w
        pltpu.sync_copy(idx_hbm.at[pl.ds(base, b_per_w)], idx_v)
        pltpu.async_copy(table_hbm.at[idx_v], rows_v, sem).wait()
        pltpu.sync_copy(rows_v, out_hbm.at[pl.ds(base, b_per_w)])

    return gather


def kernel(x, W):
    b, hw, d = x.shape
    kk = W.shape[0]
    m = b * hw
    x2 = x.reshape(m, d)
    idx_blk, dev_blk = _argmin_call(x2, W)
    idx = idx_blk.reshape(m)
    symbols = _make_gather(kk, d, m)(W, idx)
    out = symbols.reshape(b, hw, d)
    deviation = (jnp.sum(dev_blk) / jnp.float32(m * d)) * jnp.float32(1.25)
    return (out, deviation)


# speedup vs baseline: 1.1452x; 1.0340x over previous
import functools

import jax
import jax.numpy as jnp
from jax import lax
from jax.experimental import pallas as pl
from jax.experimental.pallas import tpu as pltpu
from jax.experimental.pallas import tpu_sc as plsc

BM = 1024
BK = 2048
BIG = 2 ** 30


def _argmin_body(x_ref, w_ref, idx_ref, dev_ref, wn_ref):
    i = pl.program_id(0)
    k_total = w_ref.shape[0]

    @pl.when(i == 0)
    def _():
        w = w_ref[...]
        wn_ref[...] = jnp.sum(w ** 2, axis=-1).reshape(1, k_total)

    x = x_ref[...]
    xn = jnp.sum(x ** 2, axis=-1, keepdims=True)
    x2 = x + x
    best = jnp.full((BM, 1), jnp.float32(jnp.inf))
    bidx = jnp.full((BM, 1), BIG, dtype=jnp.int32)
    for t in range(k_total // BK):
        wt = w_ref[t * BK:(t + 1) * BK, :]
        mm2 = lax.dot_general(x2, wt, (((1,), (1,)), ((), ())),
                              preferred_element_type=jnp.float32)
        wn = wn_ref[:, t * BK:(t + 1) * BK]
        s = (xn + wn) - mm2
        mn = jnp.min(s, axis=1, keepdims=True)
        io = lax.broadcasted_iota(jnp.int32, (BM, BK), 1) + t * BK
        ci = jnp.min(jnp.where(s == mn, io, jnp.int32(BIG)), axis=1,
                     keepdims=True)
        upd = mn < best
        best = jnp.where(upd, mn, best)
        bidx = jnp.where(upd, ci, bidx)
    idx_ref[0, :] = bidx[:, 0]

    @pl.when(i == 0)
    def _():
        dev_ref[0, 0] = jnp.float32(0.0)

    dev_ref[0, 0] += jnp.sum(best)


def _argmin_call(x2, w):
    m, d = x2.shape
    k = w.shape[0]
    nblk = m // BM
    return pl.pallas_call(
        _argmin_body,
        grid=(nblk,),
        in_specs=[
            pl.BlockSpec((BM, d), lambda i: (i, 0)),
            pl.BlockSpec((k, d), lambda i: (0, 0)),
        ],
        out_specs=[
            pl.BlockSpec((1, BM), lambda i: (0, i)),
            pl.BlockSpec(memory_space=pltpu.MemorySpace.SMEM),
        ],
        out_shape=[
            jax.ShapeDtypeStruct((1, m), jnp.int32),
            jax.ShapeDtypeStruct((1, 1), jnp.float32),
        ],
        scratch_shapes=[
            pltpu.VMEM((1, k), jnp.float32),
        ],
    )(x2, w)


def _make_gather(v, d, b):
    info = plsc.get_sparse_core_info()
    nw = info.num_cores * info.num_subcores
    b_per_w = b // nw
    mesh = plsc.VectorSubcoreMesh(core_axis_name="c", subcore_axis_name="s")

    @functools.partial(
        pl.kernel, mesh=mesh,
        out_type=jax.ShapeDtypeStruct((b, d), jnp.float32),
        scratch_types=[
            pltpu.VMEM((b_per_w,), jnp.int32),
            pltpu.VMEM((b_per_w, d), jnp.float32),
            pltpu.SemaphoreType.DMA,
        ],
    )
    def gather(table_hbm, idx_hbm, out_hbm, idx_v, rows_v, sem):
        wid = lax.axis_index("s") * info.num_cores + lax.axis_index("c")
        base = wid * b_per_w
        pltpu.sync_copy(idx_hbm.at[pl.ds(base, b_per_w)], idx_v)
        pltpu.async_copy(table_hbm.at[idx_v], rows_v, sem).wait()
        pltpu.sync_copy(rows_v, out_hbm.at[pl.ds(base, b_per_w)])

    return gather


def kernel(x, W):
    b, hw, d = x.shape
    kk = W.shape[0]
    m = b * hw
    x2 = x.reshape(m, d)
    idx_blk, dev_sum = _argmin_call(x2, W)
    idx = idx_blk.reshape(m)
    symbols = _make_gather(kk, d, m)(W, idx)
    out = symbols.reshape(b, hw, d)
    deviation = (dev_sum[0, 0] / jnp.float32(m * d)) * jnp.float32(1.25)
    return (out, deviation)
